# baseline (speedup 1.0000x reference)
import functools

import jax
import jax.numpy as jnp
from jax import lax
from jax.experimental import pallas as pl
from jax.experimental.pallas import tpu as pltpu
from jax.experimental.pallas import tpu_sc as plsc

_GAMMA = 0.1
_B = 16
_N = 256
_L = 16
_NC = 2
_NS = 16
_KB = _N // _L
_KPAD = _N + _L
_RT = 544

_XK_PAD = 0.0
_FK_PAD = 1e30


def _rank_loss_body(x_hbm, tgt_hbm, out_hbm,
                    xin_v, tgt_v,
                    xpk_v, fpk_v, xnk_v, rtab_v, acc_v,
                    sem_x, sem_t):
    row = lax.axis_index("s")
    half = lax.axis_index("c")
    wid = row * _NC + half

    cp_x = pltpu.async_copy(x_hbm.at[row], xin_v, sem_x)
    cp_t = pltpu.async_copy(tgt_hbm.at[row], tgt_v, sem_t)

    lanes = lax.broadcasted_iota(jnp.int32, (_L,), 0)
    xk_pad = jnp.full((_L,), _XK_PAD, jnp.float32)
    fk_pad = jnp.full((_L,), _FK_PAD, jnp.float32)

    def rtab_body(b, carry):
        s = lanes + b * _L
        sf = jnp.maximum(s, 1).astype(jnp.float32)
        rtab_v[pl.ds(b * _L, _L)] = 1.0 / sf
        return carry

    lax.fori_loop(0, _RT // _L, rtab_body, jnp.int32(0), unroll=4)

    def pad_body(b, carry):
        xpk_v[pl.ds(b * _L, _L)] = xk_pad
        fpk_v[pl.ds(b * _L, _L)] = fk_pad
        xnk_v[pl.ds(b * _L, _L)] = xk_pad
        return carry

    lax.fori_loop(0, _KPAD // _L, pad_body, jnp.int32(0))

    cp_x.wait()
    cp_t.wait()

    def cmp_k(b, cnts):
        cnt_p, cnt_n = cnts
        tg = tgt_v[pl.ds(b * _L, _L)]
        pos_b = tg != 0
        neg_b = tg == 0
        xb = 1.0 / (1.0 + jnp.exp(-xin_v[pl.ds(b * _L, _L)]))
        fb = (lanes + (b * _L + 1)).astype(jnp.float32)
        pref_p = plsc.cumsum(pos_b.astype(jnp.int32))
        pref_n = plsc.cumsum(neg_b.astype(jnp.int32))
        plsc.store_scatter(xpk_v, [cnt_p + pref_p - 1], xb, mask=pos_b)
        plsc.store_scatter(fpk_v, [cnt_p + pref_p - 1], fb, mask=pos_b)
        plsc.store_scatter(xnk_v, [cnt_n + pref_n - 1], xb, mask=neg_b)
        np_b = pref_p[_L - 1]
        return cnt_p + np_b, cnt_n + (_L - np_b)

    cnt_p, cnt_n = lax.fori_loop(0, _KB, cmp_k, (jnp.int32(0), jnp.int32(0)))

    njb = (cnt_p + (_L - 1)) // _L
    nnb = (cnt_n + (_L - 1)) // _L
    ntj = jnp.maximum(njb - half + 1, 0) // 2

    zero = jnp.zeros((_L,), jnp.float32)

    def t1(t, acc):
        jb = 2 * t + half
        jbase = jb * _L
        xjv = xpk_v[pl.ds(jbase, _L)]
        fjv = fpk_v[pl.ds(jbase, _L)]

        ikv = lanes + jbase
        fkid = jnp.minimum(fjv, 287.0).astype(jnp.int32)
        for lane in range(_L):
            xj = xjv[lane]
            fj = fjv[lane]
            fji = jnp.minimum(fj, 287.0).astype(jnp.int32)
            u = (fj - fjv) * (xj - xjv)
            w = plsc.load_gather(rtab_v, [fkid + fji])
            m = (ikv > (jbase + lane)).astype(jnp.float32)
            acc = acc + jnp.maximum(u, 0.0) * w * m

        xjs = [xjv[lane] for lane in range(_L)]
        fjs = [fjv[lane] for lane in range(_L)]
        fjis = [jnp.minimum(f, 287.0).astype(jnp.int32) for f in fjs]

        def kb1(kb, a, xjs=xjs, fjs=fjs, fjis=fjis):
            xk = xpk_v[pl.ds(kb * _L, _L)]
            fk = fpk_v[pl.ds(kb * _L, _L)]
            fki = jnp.minimum(fk, 287.0).astype(jnp.int32)
            for lane in range(_L):
                u = (fjs[lane] - fk) * (xjs[lane] - xk)
                w = plsc.load_gather(rtab_v, [fki + fjis[lane]])
                a = a + jnp.maximum(u, 0.0) * w
            return a

        return lax.fori_loop(jb + 1, njb, kb1, acc)

    acc1 = lax.fori_loop(0, ntj, t1, zero) * 2.0

    def t2(t, acc):
        jb = 2 * t + half
        jbase = jb * _L
        xjv = jnp.where(lanes + jbase < cnt_p, xpk_v[pl.ds(jbase, _L)],
                        jnp.float32(2.0))

        def kb2(kb, a, xjv=xjv):
            xk = xnk_v[pl.ds(kb * _L, _L)]
            for lane in range(_L):
                a = a + jnp.maximum(xk - xjv[lane], 0.0)
            return a

        return lax.fori_loop(0, nnb, kb2, acc)

    acc2 = lax.fori_loop(0, ntj, t2, zero)

    acc_v[...] = acc1 + jnp.float32(_GAMMA) * acc2
    pltpu.sync_copy(acc_v, out_hbm.at[wid])


def kernel(input, target, freq):
    del freq
    x = input.astype(jnp.float32)
    tgt = target.astype(jnp.int32)
    mesh = plsc.VectorSubcoreMesh(core_axis_name="c", subcore_axis_name="s")
    run = functools.partial(
        pl.kernel,
        mesh=mesh,
        out_type=jax.ShapeDtypeStruct((_NC * _NS, _L), jnp.float32),
        compiler_params=pltpu.CompilerParams(needs_layout_passes=False),
        scratch_types=[
            pltpu.VMEM((_N,), jnp.float32),
            pltpu.VMEM((_N,), jnp.int32),
            pltpu.VMEM((_KPAD,), jnp.float32),
            pltpu.VMEM((_KPAD,), jnp.float32),
            pltpu.VMEM((_KPAD,), jnp.float32),
            pltpu.VMEM((_RT,), jnp.float32),
            pltpu.VMEM((_L,), jnp.float32),
            pltpu.SemaphoreType.DMA,
            pltpu.SemaphoreType.DMA,
        ],
    )(_rank_loss_body)
    partials = run(x, tgt)
    return jnp.sum(partials) / jnp.float32(_B)


# speedup vs baseline: 1.0026x; 1.0026x over previous
import functools

import jax
import jax.numpy as jnp
from jax import lax
from jax.experimental import pallas as pl
from jax.experimental.pallas import tpu as pltpu
from jax.experimental.pallas import tpu_sc as plsc

_GAMMA = 0.1
_B = 16
_N = 256
_L = 16
_NC = 2
_NS = 16
_KB = _N // _L
_KPAD = _N + _L
_RT = 576

_XK_PAD = 0.0
_FK_PAD = 1e30


def _rank_loss_body(x_hbm, tgt_hbm, out_hbm,
                    xin_v, tgt_v,
                    xpk_v, fpk_v, xnk_v, rtab_v, acc_v,
                    sem_x, sem_t):
    row = lax.axis_index("s")
    half = lax.axis_index("c")
    wid = row * _NC + half

    cp_x = pltpu.async_copy(x_hbm.at[row], xin_v, sem_x)
    cp_t = pltpu.async_copy(tgt_hbm.at[row], tgt_v, sem_t)

    lanes = lax.broadcasted_iota(jnp.int32, (_L,), 0)
    xk_pad = jnp.full((_L,), _XK_PAD, jnp.float32)
    fk_pad = jnp.full((_L,), _FK_PAD, jnp.float32)

    def rtab_body(b, carry):
        s = lanes + b * _L
        sf = jnp.maximum(s, 1).astype(jnp.float32)
        rtab_v[pl.ds(b * _L, _L)] = 1.0 / sf
        return carry

    lax.fori_loop(0, _RT // _L, rtab_body, jnp.int32(0), unroll=4)

    def pad_body(b, carry):
        xpk_v[pl.ds(b * _L, _L)] = xk_pad
        fpk_v[pl.ds(b * _L, _L)] = fk_pad
        xnk_v[pl.ds(b * _L, _L)] = xk_pad
        return carry

    lax.fori_loop(0, _KPAD // _L, pad_body, jnp.int32(0))

    cp_x.wait()
    cp_t.wait()

    def cmp_k(b, cnts):
        cnt_p, cnt_n = cnts
        tg = tgt_v[pl.ds(b * _L, _L)]
        pos_b = tg != 0
        neg_b = tg == 0
        xb = 1.0 / (1.0 + jnp.exp(-xin_v[pl.ds(b * _L, _L)]))
        fb = (lanes + (b * _L + 1)).astype(jnp.float32)
        pref_p = plsc.cumsum(pos_b.astype(jnp.int32))
        pref_n = plsc.cumsum(neg_b.astype(jnp.int32))
        plsc.store_scatter(xpk_v, [cnt_p + pref_p - 1], xb, mask=pos_b)
        plsc.store_scatter(fpk_v, [cnt_p + pref_p - 1], fb, mask=pos_b)
        plsc.store_scatter(xnk_v, [cnt_n + pref_n - 1], xb, mask=neg_b)
        np_b = pref_p[_L - 1]
        return cnt_p + np_b, cnt_n + (_L - np_b)

    cnt_p, cnt_n = lax.fori_loop(0, _KB, cmp_k, (jnp.int32(0), jnp.int32(0)))

    njb = (cnt_p + (_L - 1)) // _L
    nnb = (cnt_n + (_L - 1)) // _L
    ntj = jnp.maximum(njb - half + 1, 0) // 2

    zero = jnp.zeros((_L,), jnp.float32)

    def t1(t, acc):
        jb = 2 * t + half
        jbase = jb * _L
        xjv = xpk_v[pl.ds(jbase, _L)]
        fjv = fpk_v[pl.ds(jbase, _L)]

        ikv = lanes + jbase
        fkid = jnp.minimum(fjv, 287.0).astype(jnp.int32)
        for lane in range(_L):
            xj = xjv[lane]
            fj = fjv[lane]
            fji = jnp.minimum(fj, 287.0).astype(jnp.int32)
            u = (fj - fjv) * (xj - xjv)
            w = plsc.load_gather(rtab_v, [fkid + fji])
            m = (ikv > (jbase + lane)).astype(jnp.float32)
            acc = acc + jnp.maximum(u, 0.0) * w * m

        def kb1(kb, a, xjv=xjv, fjv=fjv):
            xk = xpk_v[pl.ds(kb * _L, _L)]
            fk = fpk_v[pl.ds(kb * _L, _L)]
            fki = jnp.minimum(fk, 287.0).astype(jnp.int32)
            for lane in range(_L):
                xj = xjv[lane]
                fj = fjv[lane]
                fji = jnp.minimum(fj, 287.0).astype(jnp.int32)
                u = (fj - fk) * (xj - xk)
                w = plsc.load_gather(rtab_v, [fki + fji])
                a = a + jnp.maximum(u, 0.0) * w
            return a

        return lax.fori_loop(jb + 1, njb, kb1, acc)

    acc1 = lax.fori_loop(0, ntj, t1, zero) * 2.0

    def t2(t, acc):
        jb = 2 * t + half
        jbase = jb * _L
        xjv = jnp.where(lanes + jbase < cnt_p, xpk_v[pl.ds(jbase, _L)],
                        jnp.float32(2.0))

        def kb2(kb, a, xjv=xjv):
            xk = xnk_v[pl.ds(kb * _L, _L)]
            for lane in range(_L):
                a = a + jnp.maximum(xk - xjv[lane], 0.0)
            return a

        return lax.fori_loop(0, nnb, kb2, acc)

    acc2 = lax.fori_loop(0, ntj, t2, zero)

    acc_v[...] = acc1 + jnp.float32(_GAMMA) * acc2
    pltpu.sync_copy(acc_v, out_hbm.at[wid])


def kernel(input, target, freq):
    del freq
    x = input.astype(jnp.float32)
    tgt = target.astype(jnp.int32)
    mesh = plsc.VectorSubcoreMesh(core_axis_name="c", subcore_axis_name="s")
    run = functools.partial(
        pl.kernel,
        mesh=mesh,
        out_type=jax.ShapeDtypeStruct((_NC * _NS, _L), jnp.float32),
        compiler_params=pltpu.CompilerParams(needs_layout_passes=False),
        scratch_types=[
            pltpu.VMEM((_N,), jnp.float32),
            pltpu.VMEM((_N,), jnp.int32),
            pltpu.VMEM((_KPAD,), jnp.float32),
            pltpu.VMEM((_KPAD,), jnp.float32),
            pltpu.VMEM((_KPAD,), jnp.float32),
            pltpu.VMEM((_RT,), jnp.float32),
            pltpu.VMEM((_L,), jnp.float32),
            pltpu.SemaphoreType.DMA,
            pltpu.SemaphoreType.DMA,
        ],
    )(_rank_loss_body)
    partials = run(x, tgt)
    return jnp.sum(partials) / jnp.float32(_B)
